# baseline (speedup 1.0000x reference)
import functools

import jax
import jax.numpy as jnp
from jax import lax
from jax.experimental import pallas as pl
from jax.experimental.pallas import tpu as pltpu
from jax.experimental.pallas import tpu_sc as plsc

N = 10000
E = 320000
D = 128
NSTEPS = 1000

NC = 2
NS = 16
L = 16
NW = NC * NS
EPW = E // NW
CHUNK = 80
NCHUNK = EPW // CHUNK
ROWS_PT = 10240 // NS
NPAD = 10240
GPT = NPAD // NW
GCHUNK = 80
SCCH = 25

_f32 = jnp.float32
_INTERPRET = False


def _stage0_body(x_ref, y_ref, W0, b0, W1, b1, cW1, cb1, cW2, cW3, cb3,
                 outx, a0, b0c, c0):
    x = x_ref[...]
    y = y_ref[...]
    h = jnp.maximum(jnp.dot(x, W0[...], preferred_element_type=_f32) + b0[...], 0.0)
    outx[...] = jnp.dot(h, W1[...], preferred_element_type=_f32) + b1[...]
    a0[...] = jnp.dot(y, cW1[...], preferred_element_type=_f32) + cb1[...]
    b0c[...] = jnp.dot(y, cW2[...], preferred_element_type=_f32)
    c0[...] = jnp.dot(y, cW3[...], preferred_element_type=_f32) + cb3[...]


def _stage1_body(a0, c0, aggA, aggB, dwA, dwB, cW1, cb1, cW2, cW3, cb3,
                 a1, b1c, c1):
    degw = (dwA[...] + dwB[...])[:, 0:1]
    g = jnp.maximum(a0[...] * degw - (aggA[...] + aggB[...]) + c0[...], 0.0)
    a1[...] = jnp.dot(g, cW1[...], preferred_element_type=_f32) + cb1[...]
    b1c[...] = jnp.dot(g, cW2[...], preferred_element_type=_f32)
    c1[...] = jnp.dot(g, cW3[...], preferred_element_type=_f32) + cb3[...]


def _stage2_body(outx, gamma, a1, c1, aggA, aggB, dwA, dwB, out):
    degw = (dwA[...] + dwB[...])[:, 0:1]
    co = a1[...] * degw - (aggA[...] + aggB[...]) + c1[...]
    out[...] = jnp.maximum(outx[...] * (gamma[...] + co), 0.0)


_R = 1000
_GRID = (N // _R,)


def _row_spec():
    return pl.BlockSpec((_R, D), lambda i: (i, 0))


def _full_spec(r, c):
    return pl.BlockSpec((r, c), lambda i: (0, 0))


def _dw_spec():
    return pl.BlockSpec((_R, L), lambda i: (i, 0))


def _sc_edge_body(brows, src2, dst2, w2, tpad, ttab,
                  aggA, aggB, dwA, dwB, gamma,
                  agg_sh, dw_sh, isb, idb, wvb, rowsA, rowsB,
                  idx_g, dwv, dwe, sem, sem2,
                  *, with_degw_gamma):
    cid = lax.axis_index("c")
    sid = lax.axis_index("s")
    wid = sid * NC + cid

    for e in range(CHUNK):
        for q in range(D // L):
            rowsA[e, pl.ds(q * L, L)] = jnp.zeros((L,), _f32)
    for k in range(ROWS_PT // CHUNK):
        rsl = pl.ds(sid * ROWS_PT + k * CHUNK, CHUNK)
        pltpu.sync_copy(rowsA, agg_sh.at[rsl])
    dsl = pl.ds(sid * ROWS_PT, ROWS_PT)
    if with_degw_gamma:
        for v in range(ROWS_PT // L):
            dwv[pl.ds(v * L, L)] = jnp.zeros((L,), _f32)
        pltpu.sync_copy(dwv, dw_sh.at[dsl])
    plsc.subcore_barrier()

    cbase = wid * NCHUNK

    def scale(rows_ref, jrow):
        for g in range(CHUNK // L):
            wgroup = wvb[jrow, pl.ds(g * L, L)]
            for e16 in range(L):
                e = g * L + e16
                wsp = jnp.broadcast_to(wgroup[e16], (L,))
                for q in range(D // L):
                    sl = pl.ds(q * L, L)
                    rows_ref[e, sl] = rows_ref[e, sl] * wsp

    def scat(rows_ref, jrow):
        if with_degw_gamma:
            pltpu.sync_copy(wvb.at[jrow], dw_sh.at[idb.at[jrow]], add=True)
        pltpu.sync_copy(rows_ref, agg_sh.at[idb.at[jrow]], add=True)

    def scat_async(rows_ref, jrow):
        if with_degw_gamma:
            pltpu.sync_copy(wvb.at[jrow], dw_sh.at[idb.at[jrow]], add=True)
        pltpu.async_copy(rows_ref, agg_sh.at[idb.at[jrow]], sem2, add=True)

    def scat_wait(rows_ref):
        pltpu.make_async_copy(brows.at[isb.at[0]], rows_ref, sem2).wait()

    def fire(jrow, rows_ref):
        pltpu.async_copy(brows.at[isb.at[jrow]], rows_ref, sem)

    def drain(rows_ref):
        pltpu.make_async_copy(brows.at[isb.at[0]], rows_ref, sem).wait()

    def superchunk(sc_i, carry):
        rbase = cbase + sc_i * SCCH
        pltpu.sync_copy(src2.at[pl.ds(rbase, SCCH)], isb)
        pltpu.sync_copy(dst2.at[pl.ds(rbase, SCCH)], idb)
        pltpu.sync_copy(w2.at[pl.ds(rbase, SCCH)], wvb)
        fire(0, rowsA)

        def pair(j2, c2):
            j0 = 2 * j2
            drain(rowsA)
            fire(j0 + 1, rowsB)
            scale(rowsA, j0)
            scat_async(rowsA, j0)
            drain(rowsB)
            scale(rowsB, j0 + 1)
            scat_wait(rowsA)
            fire(j0 + 2, rowsA)
            scat(rowsB, j0 + 1)
            return c2

        lax.fori_loop(0, (SCCH - 1) // 2, pair, 0)
        drain(rowsA)
        scale(rowsA, SCCH - 1)
        scat(rowsA, SCCH - 1)
        return carry

    lax.fori_loop(0, NCHUNK // SCCH, superchunk, 0)

    if with_degw_gamma:
        gbase = wid * GPT
        for gj in range(GPT // GCHUNK):
            gb = pl.multiple_of(gbase + gj * GCHUNK, GCHUNK)
            pltpu.sync_copy(tpad.at[pl.ds(gb, GCHUNK)], idx_g)
            pltpu.async_copy(ttab.at[idx_g], rowsA, sem).wait()
            pltpu.sync_copy(rowsA, gamma.at[pl.ds(gb, GCHUNK)])

    plsc.subcore_barrier()

    for k in range(ROWS_PT // CHUNK):
        rsl = pl.ds(sid * ROWS_PT + k * CHUNK, CHUNK)
        pltpu.sync_copy(agg_sh.at[rsl], rowsA)

        @pl.when(cid == 0)
        def _():
            pltpu.sync_copy(rowsA, aggA.at[rsl])

        @pl.when(cid == 1)
        def _():
            pltpu.sync_copy(rowsA, aggB.at[rsl])

    if with_degw_gamma:
        pltpu.sync_copy(dw_sh.at[dsl], dwv)
        for v in range(ROWS_PT // L):
            g16 = dwv[pl.ds(v * L, L)]
            for l in range(L):
                dwe[v * L + l, pl.ds(0, L)] = jnp.broadcast_to(g16[l], (L,))

        @pl.when(cid == 0)
        def _():
            pltpu.sync_copy(dwe, dwA.at[dsl])

        @pl.when(cid == 1)
        def _():
            pltpu.sync_copy(dwe, dwB.at[dsl])


def _sc_layer0(brows, src2, dst2, w2, tpad, ttab,
               aggA, aggB, dwA, dwB, gamma,
               agg_sh, dw_sh, isb, idb, wvb, rowsA, rowsB, idx_g, dwv, dwe,
               sem, sem2):
    _sc_edge_body(brows, src2, dst2, w2, tpad, ttab,
                  aggA, aggB, dwA, dwB, gamma,
                  agg_sh, dw_sh, isb, idb, wvb, rowsA, rowsB,
                  idx_g, dwv, dwe, sem, sem2,
                  with_degw_gamma=True)


def _sc_layer1(brows, src2, dst2, w2,
               aggA, aggB,
               agg_sh, isb, idb, wvb, rowsA, rowsB, sem, sem2):
    _sc_edge_body(brows, src2, dst2, w2, None, None,
                  aggA, aggB, None, None, None,
                  agg_sh, None, isb, idb, wvb, rowsA, rowsB,
                  None, None, None, sem, sem2,
                  with_degw_gamma=False)


_SC_MESH = dict(core_axis_name="c", subcore_axis_name="s")


def _make_sc_l0():
    return pl.kernel(
        _sc_layer0,
        out_type=(
            jax.ShapeDtypeStruct((NPAD, D), _f32),
            jax.ShapeDtypeStruct((NPAD, D), _f32),
            jax.ShapeDtypeStruct((NPAD, L), _f32),
            jax.ShapeDtypeStruct((NPAD, L), _f32),
            jax.ShapeDtypeStruct((NPAD, D), _f32),
        ),
        mesh=plsc.VectorSubcoreMesh(**_SC_MESH),
        compiler_params=pltpu.CompilerParams(use_tc_tiling_on_sc=False),
        interpret=_INTERPRET,
        scratch_types=[
            pltpu.VMEM_SHARED((NPAD, D), _f32),
            pltpu.VMEM_SHARED((NPAD,), _f32),
            pltpu.VMEM((SCCH, CHUNK), jnp.int32),
            pltpu.VMEM((SCCH, CHUNK), jnp.int32),
            pltpu.VMEM((SCCH, CHUNK), _f32),
            pltpu.VMEM((CHUNK, D), _f32),
            pltpu.VMEM((CHUNK, D), _f32),
            pltpu.VMEM((CHUNK,), jnp.int32),
            pltpu.VMEM((ROWS_PT,), _f32),
            pltpu.VMEM((ROWS_PT, L), _f32),
            pltpu.SemaphoreType.DMA,
            pltpu.SemaphoreType.DMA,
        ],
    )


def _make_sc_l1():
    return pl.kernel(
        _sc_layer1,
        out_type=(
            jax.ShapeDtypeStruct((NPAD, D), _f32),
            jax.ShapeDtypeStruct((NPAD, D), _f32),
        ),
        mesh=plsc.VectorSubcoreMesh(**_SC_MESH),
        compiler_params=pltpu.CompilerParams(use_tc_tiling_on_sc=False),
        interpret=_INTERPRET,
        scratch_types=[
            pltpu.VMEM_SHARED((NPAD, D), _f32),
            pltpu.VMEM((SCCH, CHUNK), jnp.int32),
            pltpu.VMEM((SCCH, CHUNK), jnp.int32),
            pltpu.VMEM((SCCH, CHUNK), _f32),
            pltpu.VMEM((CHUNK, D), _f32),
            pltpu.VMEM((CHUNK, D), _f32),
            pltpu.SemaphoreType.DMA,
            pltpu.SemaphoreType.DMA,
        ],
    )


def kernel(x, t, y, edge_index, edge_weight,
           mlp_W0, mlp_b0, mlp_W1, mlp_b1, time_table,
           conv0_W1, conv0_b1, conv0_W2, conv0_W3, conv0_b3,
           conv1_W1, conv1_b1, conv1_W2, conv1_W3, conv1_b3):
    src = edge_index[0].astype(jnp.int32)
    dst = edge_index[1].astype(jnp.int32)
    w = edge_weight
    tpad = jnp.concatenate([t.astype(jnp.int32),
                            jnp.zeros((NPAD - N,), jnp.int32)])

    b2 = lambda b: b.reshape(1, D)

    stage0 = pl.pallas_call(
        _stage0_body,
        interpret=_INTERPRET,
        grid=_GRID,
        in_specs=[_row_spec(), _row_spec(),
                  _full_spec(D, D), _full_spec(1, D), _full_spec(D, D), _full_spec(1, D),
                  _full_spec(D, D), _full_spec(1, D), _full_spec(D, D),
                  _full_spec(D, D), _full_spec(1, D)],
        out_specs=[_row_spec()] * 4,
        out_shape=[jax.ShapeDtypeStruct((N, D), _f32)] * 4,
    )
    outx, a0, b0c, c0 = stage0(x, y, mlp_W0, b2(mlp_b0), mlp_W1, b2(mlp_b1),
                               conv0_W1, b2(conv0_b1), conv0_W2,
                               conv0_W3, b2(conv0_b3))

    src2 = src.reshape(E // CHUNK, CHUNK)
    dst2 = dst.reshape(E // CHUNK, CHUNK)
    w2 = w.reshape(E // CHUNK, CHUNK)
    aggA0, aggB0, dwA, dwB, gamma = _make_sc_l0()(
        b0c, src2, dst2, w2, tpad, time_table)

    stage1 = pl.pallas_call(
        _stage1_body,
        interpret=_INTERPRET,
        grid=_GRID,
        in_specs=[_row_spec(), _row_spec(), _row_spec(), _row_spec(),
                  _dw_spec(), _dw_spec(),
                  _full_spec(D, D), _full_spec(1, D), _full_spec(D, D),
                  _full_spec(D, D), _full_spec(1, D)],
        out_specs=[_row_spec()] * 3,
        out_shape=[jax.ShapeDtypeStruct((N, D), _f32)] * 3,
    )
    a1, b1c, c1 = stage1(a0, c0, aggA0, aggB0, dwA, dwB,
                         conv1_W1, b2(conv1_b1), conv1_W2,
                         conv1_W3, b2(conv1_b3))

    aggA1, aggB1 = _make_sc_l1()(b1c, src2, dst2, w2)

    stage2 = pl.pallas_call(
        _stage2_body,
        interpret=_INTERPRET,
        grid=_GRID,
        in_specs=[_row_spec(), _row_spec(), _row_spec(), _row_spec(),
                  _row_spec(), _row_spec(), _dw_spec(), _dw_spec()],
        out_specs=_row_spec(),
        out_shape=jax.ShapeDtypeStruct((N, D), _f32),
    )
    out = stage2(outx, gamma, a1, c1, aggA1, aggB1, dwA, dwB)
    return out


# speedup vs baseline: 1.0950x; 1.0950x over previous
import jax
import jax.numpy as jnp
from jax import lax
from jax.experimental import pallas as pl
from jax.experimental.pallas import tpu as pltpu
from jax.experimental.pallas import tpu_sc as plsc

N = 10000
E = 320000
D = 128
NSTEPS = 1000

NC = 2
NS = 16
L = 16
NW = NC * NS
EPW = E // NW
CHUNK = 80
NCHUNK = EPW // CHUNK
ROWS_PT = 10240 // NS
NPAD = 10240
GPT = NPAD // NW
GCHUNK = 80
SCCH = 25

_f32 = jnp.float32


def _stage0_body(x_ref, y_ref, W0, b0, W1, b1, cW1, cb1, cW2, cW3, cb3,
                 outx, a0, b0c, c0):
    x = x_ref[...]
    y = y_ref[...]
    h = jnp.maximum(jnp.dot(x, W0[...], preferred_element_type=_f32) + b0[...], 0.0)
    outx[...] = jnp.dot(h, W1[...], preferred_element_type=_f32) + b1[...]
    a0[...] = jnp.dot(y, cW1[...], preferred_element_type=_f32) + cb1[...]
    b0c[...] = jnp.dot(y, cW2[...], preferred_element_type=_f32)
    c0[...] = jnp.dot(y, cW3[...], preferred_element_type=_f32) + cb3[...]


def _stage1_body(a0, c0, aggA, aggB, dwA, dwB, cW1, cb1, cW2, cW3, cb3,
                 a1, b1c, c1):
    degw = (dwA[...] + dwB[...])[:, 0:1]
    g = jnp.maximum(a0[...] * degw - (aggA[...] + aggB[...]) + c0[...], 0.0)
    a1[...] = jnp.dot(g, cW1[...], preferred_element_type=_f32) + cb1[...]
    b1c[...] = jnp.dot(g, cW2[...], preferred_element_type=_f32)
    c1[...] = jnp.dot(g, cW3[...], preferred_element_type=_f32) + cb3[...]


def _stage2_body(outx, gamma, a1, c1, aggA, aggB, dwA, dwB, out):
    degw = (dwA[...] + dwB[...])[:, 0:1]
    co = a1[...] * degw - (aggA[...] + aggB[...]) + c1[...]
    out[...] = jnp.maximum(outx[...] * (gamma[...] + co), 0.0)


_R = 1000
_GRID = (N // _R,)


def _row_spec():
    return pl.BlockSpec((_R, D), lambda i: (i, 0))


def _full_spec(r, c):
    return pl.BlockSpec((r, c), lambda i: (0, 0))


def _dw_spec():
    return pl.BlockSpec((_R, L), lambda i: (i, 0))


def _sc_edge_body(brows, src2, dst2, w2, tpad, ttab,
                  aggA, aggB, dwA, dwB, gamma,
                  agg_sh, dw_sh, isb, idb, wvb, rowsA, rowsB,
                  idx_g, dwv, dwe, sem,
                  *, with_degw_gamma):
    cid = lax.axis_index("c")
    sid = lax.axis_index("s")
    wid = sid * NC + cid

    for e in range(CHUNK):
        for q in range(D // L):
            rowsA[e, pl.ds(q * L, L)] = jnp.zeros((L,), _f32)
    for k in range(ROWS_PT // CHUNK):
        rsl = pl.ds(sid * ROWS_PT + k * CHUNK, CHUNK)
        pltpu.sync_copy(rowsA, agg_sh.at[rsl])
    dsl = pl.ds(sid * ROWS_PT, ROWS_PT)
    if with_degw_gamma:
        for v in range(ROWS_PT // L):
            dwv[pl.ds(v * L, L)] = jnp.zeros((L,), _f32)
        pltpu.sync_copy(dwv, dw_sh.at[dsl])
    plsc.subcore_barrier()

    cbase = wid * NCHUNK

    def scale(rows_ref, jrow):
        for g in range(CHUNK // L):
            wgroup = wvb[jrow, pl.ds(g * L, L)]
            for e16 in range(L):
                e = g * L + e16
                wsp = jnp.broadcast_to(wgroup[e16], (L,))
                for q in range(D // L):
                    sl = pl.ds(q * L, L)
                    rows_ref[e, sl] = rows_ref[e, sl] * wsp

    def scat(rows_ref, jrow):
        if with_degw_gamma:
            pltpu.sync_copy(wvb.at[jrow], dw_sh.at[idb.at[jrow]], add=True)
        pltpu.sync_copy(rows_ref, agg_sh.at[idb.at[jrow]], add=True)

    def fire(jrow, rows_ref):
        pltpu.async_copy(brows.at[isb.at[jrow]], rows_ref, sem)

    def drain(rows_ref):
        pltpu.make_async_copy(brows.at[isb.at[0]], rows_ref, sem).wait()

    def superchunk(sc_i, carry):
        rbase = cbase + sc_i * SCCH
        pltpu.sync_copy(src2.at[pl.ds(rbase, SCCH)], isb)
        pltpu.sync_copy(dst2.at[pl.ds(rbase, SCCH)], idb)
        pltpu.sync_copy(w2.at[pl.ds(rbase, SCCH)], wvb)
        fire(0, rowsA)

        def pair(j2, c2):
            j0 = 2 * j2
            drain(rowsA)
            fire(j0 + 1, rowsB)
            scale(rowsA, j0)
            scat(rowsA, j0)
            drain(rowsB)
            fire(j0 + 2, rowsA)
            scale(rowsB, j0 + 1)
            scat(rowsB, j0 + 1)
            return c2

        lax.fori_loop(0, (SCCH - 1) // 2, pair, 0)
        drain(rowsA)
        scale(rowsA, SCCH - 1)
        scat(rowsA, SCCH - 1)
        return carry

    lax.fori_loop(0, NCHUNK // SCCH, superchunk, 0)

    if with_degw_gamma:
        gbase = wid * GPT
        for gj in range(GPT // GCHUNK):
            gb = pl.multiple_of(gbase + gj * GCHUNK, GCHUNK)
            pltpu.sync_copy(tpad.at[pl.ds(gb, GCHUNK)], idx_g)
            pltpu.async_copy(ttab.at[idx_g], rowsA, sem).wait()
            pltpu.sync_copy(rowsA, gamma.at[pl.ds(gb, GCHUNK)])

    plsc.subcore_barrier()

    for k in range(ROWS_PT // CHUNK):
        rsl = pl.ds(sid * ROWS_PT + k * CHUNK, CHUNK)
        pltpu.sync_copy(agg_sh.at[rsl], rowsA)

        @pl.when(cid == 0)
        def _():
            pltpu.sync_copy(rowsA, aggA.at[rsl])

        @pl.when(cid == 1)
        def _():
            pltpu.sync_copy(rowsA, aggB.at[rsl])

    if with_degw_gamma:
        pltpu.sync_copy(dw_sh.at[dsl], dwv)
        for v in range(ROWS_PT // L):
            g16 = dwv[pl.ds(v * L, L)]
            for l in range(L):
                dwe[v * L + l, pl.ds(0, L)] = jnp.broadcast_to(g16[l], (L,))

        @pl.when(cid == 0)
        def _():
            pltpu.sync_copy(dwe, dwA.at[dsl])

        @pl.when(cid == 1)
        def _():
            pltpu.sync_copy(dwe, dwB.at[dsl])


def _sc_layer0(brows, src2, dst2, w2, tpad, ttab,
               aggA, aggB, dwA, dwB, gamma,
               agg_sh, dw_sh, isb, idb, wvb, rowsA, rowsB, idx_g, dwv, dwe,
               sem):
    _sc_edge_body(brows, src2, dst2, w2, tpad, ttab,
                  aggA, aggB, dwA, dwB, gamma,
                  agg_sh, dw_sh, isb, idb, wvb, rowsA, rowsB,
                  idx_g, dwv, dwe, sem,
                  with_degw_gamma=True)


def _sc_layer1(brows, src2, dst2, w2,
               aggA, aggB,
               agg_sh, isb, idb, wvb, rowsA, rowsB, sem):
    _sc_edge_body(brows, src2, dst2, w2, None, None,
                  aggA, aggB, None, None, None,
                  agg_sh, None, isb, idb, wvb, rowsA, rowsB,
                  None, None, None, sem,
                  with_degw_gamma=False)


_SC_MESH = dict(core_axis_name="c", subcore_axis_name="s")


def _make_sc_l0():
    return pl.kernel(
        _sc_layer0,
        out_type=(
            jax.ShapeDtypeStruct((NPAD, D), _f32),
            jax.ShapeDtypeStruct((NPAD, D), _f32),
            jax.ShapeDtypeStruct((NPAD, L), _f32),
            jax.ShapeDtypeStruct((NPAD, L), _f32),
            jax.ShapeDtypeStruct((NPAD, D), _f32),
        ),
        mesh=plsc.VectorSubcoreMesh(**_SC_MESH),
        compiler_params=pltpu.CompilerParams(use_tc_tiling_on_sc=False),
        scratch_types=[
            pltpu.VMEM_SHARED((NPAD, D), _f32),
            pltpu.VMEM_SHARED((NPAD,), _f32),
            pltpu.VMEM((SCCH, CHUNK), jnp.int32),
            pltpu.VMEM((SCCH, CHUNK), jnp.int32),
            pltpu.VMEM((SCCH, CHUNK), _f32),
            pltpu.VMEM((CHUNK, D), _f32),
            pltpu.VMEM((CHUNK, D), _f32),
            pltpu.VMEM((CHUNK,), jnp.int32),
            pltpu.VMEM((ROWS_PT,), _f32),
            pltpu.VMEM((ROWS_PT, L), _f32),
            pltpu.SemaphoreType.DMA,
        ],
    )


def _make_sc_l1():
    return pl.kernel(
        _sc_layer1,
        out_type=(
            jax.ShapeDtypeStruct((NPAD, D), _f32),
            jax.ShapeDtypeStruct((NPAD, D), _f32),
        ),
        mesh=plsc.VectorSubcoreMesh(**_SC_MESH),
        compiler_params=pltpu.CompilerParams(use_tc_tiling_on_sc=False),
        scratch_types=[
            pltpu.VMEM_SHARED((NPAD, D), _f32),
            pltpu.VMEM((SCCH, CHUNK), jnp.int32),
            pltpu.VMEM((SCCH, CHUNK), jnp.int32),
            pltpu.VMEM((SCCH, CHUNK), _f32),
            pltpu.VMEM((CHUNK, D), _f32),
            pltpu.VMEM((CHUNK, D), _f32),
            pltpu.SemaphoreType.DMA,
        ],
    )


def kernel(x, t, y, edge_index, edge_weight,
           mlp_W0, mlp_b0, mlp_W1, mlp_b1, time_table,
           conv0_W1, conv0_b1, conv0_W2, conv0_W3, conv0_b3,
           conv1_W1, conv1_b1, conv1_W2, conv1_W3, conv1_b3):
    src = edge_index[0].astype(jnp.int32)
    dst = edge_index[1].astype(jnp.int32)
    w = edge_weight
    tpad = jnp.concatenate([t.astype(jnp.int32),
                            jnp.zeros((NPAD - N,), jnp.int32)])

    b2 = lambda b: b.reshape(1, D)

    stage0 = pl.pallas_call(
        _stage0_body,
        grid=_GRID,
        in_specs=[_row_spec(), _row_spec(),
                  _full_spec(D, D), _full_spec(1, D), _full_spec(D, D), _full_spec(1, D),
                  _full_spec(D, D), _full_spec(1, D), _full_spec(D, D),
                  _full_spec(D, D), _full_spec(1, D)],
        out_specs=[_row_spec()] * 4,
        out_shape=[jax.ShapeDtypeStruct((N, D), _f32)] * 4,
    )
    outx, a0, b0c, c0 = stage0(x, y, mlp_W0, b2(mlp_b0), mlp_W1, b2(mlp_b1),
                               conv0_W1, b2(conv0_b1), conv0_W2,
                               conv0_W3, b2(conv0_b3))

    src2 = src.reshape(E // CHUNK, CHUNK)
    dst2 = dst.reshape(E // CHUNK, CHUNK)
    w2 = w.reshape(E // CHUNK, CHUNK)
    aggA0, aggB0, dwA, dwB, gamma = _make_sc_l0()(
        b0c, src2, dst2, w2, tpad, time_table)

    stage1 = pl.pallas_call(
        _stage1_body,
        grid=_GRID,
        in_specs=[_row_spec(), _row_spec(), _row_spec(), _row_spec(),
                  _dw_spec(), _dw_spec(),
                  _full_spec(D, D), _full_spec(1, D), _full_spec(D, D),
                  _full_spec(D, D), _full_spec(1, D)],
        out_specs=[_row_spec()] * 3,
        out_shape=[jax.ShapeDtypeStruct((N, D), _f32)] * 3,
    )
    a1, b1c, c1 = stage1(a0, c0, aggA0, aggB0, dwA, dwB,
                         conv1_W1, b2(conv1_b1), conv1_W2,
                         conv1_W3, b2(conv1_b3))

    aggA1, aggB1 = _make_sc_l1()(b1c, src2, dst2, w2)

    stage2 = pl.pallas_call(
        _stage2_body,
        grid=_GRID,
        in_specs=[_row_spec(), _row_spec(), _row_spec(), _row_spec(),
                  _row_spec(), _row_spec(), _dw_spec(), _dw_spec()],
        out_specs=_row_spec(),
        out_shape=jax.ShapeDtypeStruct((N, D), _f32),
    )
    out = stage2(outx, gamma, a1, c1, aggA1, aggB1, dwA, dwB)
    return out


# speedup vs baseline: 1.1115x; 1.0151x over previous
import jax
import jax.numpy as jnp
from jax import lax
from jax.experimental import pallas as pl
from jax.experimental.pallas import tpu as pltpu
from jax.experimental.pallas import tpu_sc as plsc

N = 10000
E = 320000
D = 128
NSTEPS = 1000

NC = 2
NS = 16
L = 16
NW = NC * NS
EPW = E // NW
CHUNK = 80
NCHUNK = EPW // CHUNK
ROWS_PT = 10240 // NS
NPAD = 10240
GPT = NPAD // NW
GCHUNK = 80
SCCH = 25

_f32 = jnp.float32


def _stage0a_body(y_ref, cW2, b0c):
    b0c[...] = jnp.dot(y_ref[...], cW2[...], preferred_element_type=_f32)


def _stage0b_body(x_ref, y_ref, W0, b0, W1, b1, cW1, cb1, cW3, cb3,
                  outx, a0, c0):
    x = x_ref[...]
    y = y_ref[...]
    h = jnp.maximum(jnp.dot(x, W0[...], preferred_element_type=_f32) + b0[...], 0.0)
    outx[...] = jnp.dot(h, W1[...], preferred_element_type=_f32) + b1[...]
    a0[...] = jnp.dot(y, cW1[...], preferred_element_type=_f32) + cb1[...]
    c0[...] = jnp.dot(y, cW3[...], preferred_element_type=_f32) + cb3[...]


def _stage1a_body(a0, c0, aggA, aggB, dwA, dwB, cW2, g_out, b1c):
    degw = (dwA[...] + dwB[...])[:, 0:1]
    g = jnp.maximum(a0[...] * degw - (aggA[...] + aggB[...]) + c0[...], 0.0)
    g_out[...] = g
    b1c[...] = jnp.dot(g, cW2[...], preferred_element_type=_f32)


def _stage1b_body(g_ref, cW1, cb1, cW3, cb3, a1, c1):
    g = g_ref[...]
    a1[...] = jnp.dot(g, cW1[...], preferred_element_type=_f32) + cb1[...]
    c1[...] = jnp.dot(g, cW3[...], preferred_element_type=_f32) + cb3[...]


def _stage2_body(outx, gamma, a1, c1, aggA, aggB, dwA, dwB, out):
    degw = (dwA[...] + dwB[...])[:, 0:1]
    co = a1[...] * degw - (aggA[...] + aggB[...]) + c1[...]
    out[...] = jnp.maximum(outx[...] * (gamma[...] + co), 0.0)


_R = 1000
_GRID = (N // _R,)


def _row_spec():
    return pl.BlockSpec((_R, D), lambda i: (i, 0))


def _full_spec(r, c):
    return pl.BlockSpec((r, c), lambda i: (0, 0))


def _dw_spec():
    return pl.BlockSpec((_R, L), lambda i: (i, 0))


def _sc_edge_body(brows, src2, dst2, w2, tpad, ttab,
                  aggA, aggB, dwA, dwB, gamma,
                  agg_sh, dw_sh, isb, idb, wvb, rowsA, rowsB,
                  idx_g, dwv, dwe, sem, sem2,
                  *, with_degw_gamma):
    cid = lax.axis_index("c")
    sid = lax.axis_index("s")
    wid = sid * NC + cid

    for e in range(CHUNK):
        for q in range(D // L):
            rowsA[e, pl.ds(q * L, L)] = jnp.zeros((L,), _f32)
    for k in range(ROWS_PT // CHUNK):
        rsl = pl.ds(sid * ROWS_PT + k * CHUNK, CHUNK)
        pltpu.sync_copy(rowsA, agg_sh.at[rsl])
    dsl = pl.ds(sid * ROWS_PT, ROWS_PT)
    if with_degw_gamma:
        for v in range(ROWS_PT // L):
            dwv[pl.ds(v * L, L)] = jnp.zeros((L,), _f32)
        pltpu.sync_copy(dwv, dw_sh.at[dsl])
    plsc.subcore_barrier()

    cbase = wid * NCHUNK

    def scale(rows_ref, jrow):
        for g in range(CHUNK // L):
            wgroup = wvb[jrow, pl.ds(g * L, L)]
            for e16 in range(L):
                e = g * L + e16
                wsp = jnp.broadcast_to(wgroup[e16], (L,))
                for q in range(D // L):
                    sl = pl.ds(q * L, L)
                    rows_ref[e, sl] = rows_ref[e, sl] * wsp

    def scat(rows_ref, jrow):
        if with_degw_gamma:
            pltpu.async_copy(wvb.at[jrow], dw_sh.at[idb.at[jrow]], sem2,
                             add=True)
        pltpu.sync_copy(rows_ref, agg_sh.at[idb.at[jrow]], add=True)

    def fire(jrow, rows_ref):
        pltpu.async_copy(brows.at[isb.at[jrow]], rows_ref, sem)

    def drain(rows_ref):
        pltpu.make_async_copy(brows.at[isb.at[0]], rows_ref, sem).wait()

    def superchunk(sc_i, carry):
        rbase = cbase + sc_i * SCCH
        pltpu.sync_copy(src2.at[pl.ds(rbase, SCCH)], isb)
        pltpu.sync_copy(dst2.at[pl.ds(rbase, SCCH)], idb)
        pltpu.sync_copy(w2.at[pl.ds(rbase, SCCH)], wvb)
        fire(0, rowsA)

        def pair(j2, c2):
            j0 = 2 * j2
            drain(rowsA)
            fire(j0 + 1, rowsB)
            scale(rowsA, j0)
            scat(rowsA, j0)
            drain(rowsB)
            fire(j0 + 2, rowsA)
            scale(rowsB, j0 + 1)
            scat(rowsB, j0 + 1)
            return c2

        lax.fori_loop(0, (SCCH - 1) // 2, pair, 0)
        drain(rowsA)
        scale(rowsA, SCCH - 1)
        scat(rowsA, SCCH - 1)
        if with_degw_gamma:
            for _ in range(SCCH):
                pltpu.make_async_copy(w2.at[pl.ds(0, CHUNK)], wvb.at[0],
                                      sem2).wait()
        return carry

    lax.fori_loop(0, NCHUNK // SCCH, superchunk, 0)

    if with_degw_gamma:
        gbase = wid * GPT
        for gj in range(GPT // GCHUNK):
            gb = pl.multiple_of(gbase + gj * GCHUNK, GCHUNK)
            pltpu.sync_copy(tpad.at[pl.ds(gb, GCHUNK)], idx_g)
            pltpu.async_copy(ttab.at[idx_g], rowsA, sem).wait()
            pltpu.sync_copy(rowsA, gamma.at[pl.ds(gb, GCHUNK)])

    plsc.subcore_barrier()

    for k in range(ROWS_PT // CHUNK):
        rsl = pl.ds(sid * ROWS_PT + k * CHUNK, CHUNK)
        pltpu.sync_copy(agg_sh.at[rsl], rowsA)

        @pl.when(cid == 0)
        def _():
            pltpu.sync_copy(rowsA, aggA.at[rsl])

        @pl.when(cid == 1)
        def _():
            pltpu.sync_copy(rowsA, aggB.at[rsl])

    if with_degw_gamma:
        pltpu.sync_copy(dw_sh.at[dsl], dwv)
        for v in range(ROWS_PT // L):
            g16 = dwv[pl.ds(v * L, L)]
            for l in range(L):
                dwe[v * L + l, pl.ds(0, L)] = jnp.broadcast_to(g16[l], (L,))

        @pl.when(cid == 0)
        def _():
            pltpu.sync_copy(dwe, dwA.at[dsl])

        @pl.when(cid == 1)
        def _():
            pltpu.sync_copy(dwe, dwB.at[dsl])


def _sc_layer0(brows, src2, dst2, w2, tpad, ttab,
               aggA, aggB, dwA, dwB, gamma,
               agg_sh, dw_sh, isb, idb, wvb, rowsA, rowsB, idx_g, dwv, dwe,
               sem, sem2):
    _sc_edge_body(brows, src2, dst2, w2, tpad, ttab,
                  aggA, aggB, dwA, dwB, gamma,
                  agg_sh, dw_sh, isb, idb, wvb, rowsA, rowsB,
                  idx_g, dwv, dwe, sem, sem2,
                  with_degw_gamma=True)


def _sc_layer1(brows, src2, dst2, w2,
               aggA, aggB,
               agg_sh, isb, idb, wvb, rowsA, rowsB, sem):
    _sc_edge_body(brows, src2, dst2, w2, None, None,
                  aggA, aggB, None, None, None,
                  agg_sh, None, isb, idb, wvb, rowsA, rowsB,
                  None, None, None, sem, None,
                  with_degw_gamma=False)


_SC_MESH = dict(core_axis_name="c", subcore_axis_name="s")


def _make_sc_l0():
    return pl.kernel(
        _sc_layer0,
        out_type=(
            jax.ShapeDtypeStruct((NPAD, D), _f32),
            jax.ShapeDtypeStruct((NPAD, D), _f32),
            jax.ShapeDtypeStruct((NPAD, L), _f32),
            jax.ShapeDtypeStruct((NPAD, L), _f32),
            jax.ShapeDtypeStruct((NPAD, D), _f32),
        ),
        mesh=plsc.VectorSubcoreMesh(**_SC_MESH),
        compiler_params=pltpu.CompilerParams(use_tc_tiling_on_sc=False),
        scratch_types=[
            pltpu.VMEM_SHARED((NPAD, D), _f32),
            pltpu.VMEM_SHARED((NPAD,), _f32),
            pltpu.VMEM((SCCH, CHUNK), jnp.int32),
            pltpu.VMEM((SCCH, CHUNK), jnp.int32),
            pltpu.VMEM((SCCH, CHUNK), _f32),
            pltpu.VMEM((CHUNK, D), _f32),
            pltpu.VMEM((CHUNK, D), _f32),
            pltpu.VMEM((CHUNK,), jnp.int32),
            pltpu.VMEM((ROWS_PT,), _f32),
            pltpu.VMEM((ROWS_PT, L), _f32),
            pltpu.SemaphoreType.DMA,
            pltpu.SemaphoreType.DMA,
        ],
    )


def _make_sc_l1():
    return pl.kernel(
        _sc_layer1,
        out_type=(
            jax.ShapeDtypeStruct((NPAD, D), _f32),
            jax.ShapeDtypeStruct((NPAD, D), _f32),
        ),
        mesh=plsc.VectorSubcoreMesh(**_SC_MESH),
        compiler_params=pltpu.CompilerParams(use_tc_tiling_on_sc=False),
        scratch_types=[
            pltpu.VMEM_SHARED((NPAD, D), _f32),
            pltpu.VMEM((SCCH, CHUNK), jnp.int32),
            pltpu.VMEM((SCCH, CHUNK), jnp.int32),
            pltpu.VMEM((SCCH, CHUNK), _f32),
            pltpu.VMEM((CHUNK, D), _f32),
            pltpu.VMEM((CHUNK, D), _f32),
            pltpu.SemaphoreType.DMA,
        ],
    )


def kernel(x, t, y, edge_index, edge_weight,
           mlp_W0, mlp_b0, mlp_W1, mlp_b1, time_table,
           conv0_W1, conv0_b1, conv0_W2, conv0_W3, conv0_b3,
           conv1_W1, conv1_b1, conv1_W2, conv1_W3, conv1_b3):
    src = edge_index[0].astype(jnp.int32)
    dst = edge_index[1].astype(jnp.int32)
    w = edge_weight
    tpad = jnp.concatenate([t.astype(jnp.int32),
                            jnp.zeros((NPAD - N,), jnp.int32)])

    b2 = lambda b: b.reshape(1, D)

    stage0a = pl.pallas_call(
        _stage0a_body,
        grid=_GRID,
        in_specs=[_row_spec(), _full_spec(D, D)],
        out_specs=_row_spec(),
        out_shape=jax.ShapeDtypeStruct((N, D), _f32),
    )
    b0c = stage0a(y, conv0_W2)

    src2 = src.reshape(E // CHUNK, CHUNK)
    dst2 = dst.reshape(E // CHUNK, CHUNK)
    w2 = w.reshape(E // CHUNK, CHUNK)
    aggA0, aggB0, dwA, dwB, gamma = _make_sc_l0()(
        b0c, src2, dst2, w2, tpad, time_table)

    stage0b = pl.pallas_call(
        _stage0b_body,
        grid=_GRID,
        in_specs=[_row_spec(), _row_spec(),
                  _full_spec(D, D), _full_spec(1, D), _full_spec(D, D), _full_spec(1, D),
                  _full_spec(D, D), _full_spec(1, D),
                  _full_spec(D, D), _full_spec(1, D)],
        out_specs=[_row_spec()] * 3,
        out_shape=[jax.ShapeDtypeStruct((N, D), _f32)] * 3,
    )
    outx, a0, c0 = stage0b(x, y, mlp_W0, b2(mlp_b0), mlp_W1, b2(mlp_b1),
                           conv0_W1, b2(conv0_b1), conv0_W3, b2(conv0_b3))

    stage1a = pl.pallas_call(
        _stage1a_body,
        grid=_GRID,
        in_specs=[_row_spec(), _row_spec(), _row_spec(), _row_spec(),
                  _dw_spec(), _dw_spec(), _full_spec(D, D)],
        out_specs=[_row_spec()] * 2,
        out_shape=[jax.ShapeDtypeStruct((N, D), _f32)] * 2,
    )
    g, b1c = stage1a(a0, c0, aggA0, aggB0, dwA, dwB, conv1_W2)

    aggA1, aggB1 = _make_sc_l1()(b1c, src2, dst2, w2)

    stage1b = pl.pallas_call(
        _stage1b_body,
        grid=_GRID,
        in_specs=[_row_spec(), _full_spec(D, D), _full_spec(1, D),
                  _full_spec(D, D), _full_spec(1, D)],
        out_specs=[_row_spec()] * 2,
        out_shape=[jax.ShapeDtypeStruct((N, D), _f32)] * 2,
    )
    a1, c1 = stage1b(g, conv1_W1, b2(conv1_b1), conv1_W3, b2(conv1_b3))

    stage2 = pl.pallas_call(
        _stage2_body,
        grid=_GRID,
        in_specs=[_row_spec(), _row_spec(), _row_spec(), _row_spec(),
                  _row_spec(), _row_spec(), _dw_spec(), _dw_spec()],
        out_specs=_row_spec(),
        out_shape=jax.ShapeDtypeStruct((N, D), _f32),
    )
    out = stage2(outx, gamma, a1, c1, aggA1, aggB1, dwA, dwB)
    return out
